# baseline (speedup 1.0000x reference)
import functools

import jax
import jax.numpy as jnp
from jax.experimental import pallas as pl

NUM_LEVELS = 4
K = 1024
D = 256
BLOCK_B = 512


def _rq_kernel(x_ref, cb2_ref, hi_ref, b0_ref, b1_ref, cbnb_ref,
               codes_ref, quant_ref):
    residual_t = jnp.transpose(x_ref[...])
    bb = residual_t.shape[1]
    quant_t = jnp.zeros_like(residual_t)
    iota_kt = jax.lax.broadcasted_iota(jnp.int32, (K, bb), 0)
    dn = (((1,), (0,)), ((), ()))
    for l in range(NUM_LEVELS):
        r_norm = jnp.sum(residual_t * residual_t, axis=0, keepdims=True)
        d2_t = (
            r_norm
            - jax.lax.dot_general(
                cb2_ref[l], residual_t, dn,
                preferred_element_type=jnp.float32,
            )
        ) + cbnb_ref[l]
        m = jnp.min(d2_t, axis=0, keepdims=True)
        masked = jnp.where(d2_t == m, iota_kt, K)
        codes = jnp.min(masked, axis=0)
        onehot_t = (masked == codes[None, :]).astype(jnp.bfloat16)
        q_hi = jax.lax.dot_general(hi_ref[l], onehot_t, dn,
                                   preferred_element_type=jnp.float32)
        hi_bits = jax.lax.bitcast_convert_type(q_hi, jnp.int32)
        v0 = jax.lax.dot_general(b0_ref[l], onehot_t, dn,
                                 preferred_element_type=jnp.float32)
        v1 = jax.lax.dot_general(b1_ref[l], onehot_t, dn,
                                 preferred_element_type=jnp.float32)
        low16 = (v0 + v1).astype(jnp.int32)
        q_t = jax.lax.bitcast_convert_type(hi_bits | low16, jnp.float32)
        codes_ref[l, :] = codes
        quant_t = quant_t + q_t
        residual_t = residual_t - q_t
    quant_ref[...] = jnp.transpose(quant_t)


@jax.jit
def kernel(x, codebooks):
    b, d = x.shape
    cb_norms = jnp.stack(
        [jnp.sum(codebooks[l] * codebooks[l], axis=1)
         for l in range(NUM_LEVELS)], axis=0)
    cbn_b = jnp.broadcast_to(cb_norms[:, :, None],
                             (NUM_LEVELS, K, BLOCK_B))
    cb_bytes = jax.lax.bitcast_convert_type(codebooks, jnp.uint8)
    b0_plane = cb_bytes[..., 0].astype(jnp.bfloat16).swapaxes(1, 2)
    b1_plane = (cb_bytes[..., 1].astype(jnp.float32)
                * 256.0).astype(jnp.bfloat16).swapaxes(1, 2)
    hi_plane = jax.lax.bitcast_convert_type(
        jax.lax.bitcast_convert_type(codebooks, jnp.int16)[..., 1],
        jnp.bfloat16).swapaxes(1, 2)
    grid = (b // BLOCK_B,)
    plane_spec = pl.BlockSpec((NUM_LEVELS, D, K), lambda i: (0, 0, 0))
    codes_t, quant = pl.pallas_call(
        _rq_kernel,
        grid=grid,
        in_specs=[
            pl.BlockSpec((BLOCK_B, d), lambda i: (i, 0)),
            pl.BlockSpec((NUM_LEVELS, K, D), lambda i: (0, 0, 0)),
            plane_spec,
            plane_spec,
            plane_spec,
            pl.BlockSpec((NUM_LEVELS, K, BLOCK_B), lambda i: (0, 0, 0)),
        ],
        out_specs=[
            pl.BlockSpec((NUM_LEVELS, BLOCK_B), lambda i: (0, i)),
            pl.BlockSpec((BLOCK_B, d), lambda i: (i, 0)),
        ],
        out_shape=[
            jax.ShapeDtypeStruct((NUM_LEVELS, b), jnp.int32),
            jax.ShapeDtypeStruct((b, d), jnp.float32),
        ],
    )(x, 2.0 * codebooks, hi_plane, b0_plane, b1_plane, cbn_b)
    return codes_t.T, quant


# speedup vs baseline: 1.1825x; 1.1825x over previous
import functools

import jax
import jax.numpy as jnp
from jax.experimental import pallas as pl

NUM_LEVELS = 4
K = 1024
D = 256
BLOCK_B = 1024


def _rq_kernel(x_ref, cb2_ref, hi_ref, b0_ref, b1_ref, cbnb_ref,
               codes_ref, quant_ref):
    residual_t = jnp.transpose(x_ref[...])
    bb = residual_t.shape[1]
    quant_t = jnp.zeros_like(residual_t)
    iota_kt = jax.lax.broadcasted_iota(jnp.int32, (K, bb), 0)
    dn = (((1,), (0,)), ((), ()))
    for l in range(NUM_LEVELS):
        r_norm = jnp.sum(residual_t * residual_t, axis=0, keepdims=True)
        d2_t = (
            r_norm
            - jax.lax.dot_general(
                cb2_ref[l], residual_t, dn,
                preferred_element_type=jnp.float32,
            )
        ) + cbnb_ref[l]
        m = jnp.min(d2_t, axis=0, keepdims=True)
        masked = jnp.where(d2_t == m, iota_kt, K)
        codes = jnp.min(masked, axis=0)
        onehot_t = (masked == codes[None, :]).astype(jnp.bfloat16)
        q_hi = jax.lax.dot_general(hi_ref[l], onehot_t, dn,
                                   preferred_element_type=jnp.float32)
        hi_bits = jax.lax.bitcast_convert_type(q_hi, jnp.int32)
        v0 = jax.lax.dot_general(b0_ref[l], onehot_t, dn,
                                 preferred_element_type=jnp.float32)
        v1 = jax.lax.dot_general(b1_ref[l], onehot_t, dn,
                                 preferred_element_type=jnp.float32)
        low16 = (v0 + v1).astype(jnp.int32)
        q_t = jax.lax.bitcast_convert_type(hi_bits | low16, jnp.float32)
        codes_ref[l, :] = codes
        quant_t = quant_t + q_t
        residual_t = residual_t - q_t
    quant_ref[...] = jnp.transpose(quant_t)


@jax.jit
def kernel(x, codebooks):
    b, d = x.shape
    cb_norms = jnp.stack(
        [jnp.sum(codebooks[l] * codebooks[l], axis=1)
         for l in range(NUM_LEVELS)], axis=0)
    cbn_b = jnp.broadcast_to(cb_norms[:, :, None],
                             (NUM_LEVELS, K, BLOCK_B))
    cb_bytes = jax.lax.bitcast_convert_type(codebooks, jnp.uint8)
    b0_plane = cb_bytes[..., 0].astype(jnp.bfloat16).swapaxes(1, 2)
    b1_plane = (cb_bytes[..., 1].astype(jnp.float32)
                * 256.0).astype(jnp.bfloat16).swapaxes(1, 2)
    hi_plane = jax.lax.bitcast_convert_type(
        jax.lax.bitcast_convert_type(codebooks, jnp.int16)[..., 1],
        jnp.bfloat16).swapaxes(1, 2)
    grid = (b // BLOCK_B,)
    plane_spec = pl.BlockSpec((NUM_LEVELS, D, K), lambda i: (0, 0, 0))
    codes_t, quant = pl.pallas_call(
        _rq_kernel,
        grid=grid,
        in_specs=[
            pl.BlockSpec((BLOCK_B, d), lambda i: (i, 0)),
            pl.BlockSpec((NUM_LEVELS, K, D), lambda i: (0, 0, 0)),
            plane_spec,
            plane_spec,
            plane_spec,
            pl.BlockSpec((NUM_LEVELS, K, BLOCK_B), lambda i: (0, 0, 0)),
        ],
        out_specs=[
            pl.BlockSpec((NUM_LEVELS, BLOCK_B), lambda i: (0, i)),
            pl.BlockSpec((BLOCK_B, d), lambda i: (i, 0)),
        ],
        out_shape=[
            jax.ShapeDtypeStruct((NUM_LEVELS, b), jnp.int32),
            jax.ShapeDtypeStruct((b, d), jnp.float32),
        ],
    )(x, 2.0 * codebooks, hi_plane, b0_plane, b1_plane, cbn_b)
    return codes_t.T, quant


# speedup vs baseline: 1.2055x; 1.0194x over previous
import functools

import jax
import jax.numpy as jnp
from jax.experimental import pallas as pl
from jax.experimental.pallas import tpu as pltpu

NUM_LEVELS = 4
K = 1024
D = 256
BLOCK_B = 1024


def _rq_kernel(x_ref, cb2_ref, hi_ref, b0_ref, b1_ref, cbn_ref,
               codes_ref, quant_ref, cbnb_ref):
    @pl.when(pl.program_id(0) == 0)
    def _fill():
        for l in range(NUM_LEVELS):
            cbnb_ref[l] = jnp.broadcast_to(cbn_ref[l][:, None],
                                           (K, BLOCK_B))

    residual_t = jnp.transpose(x_ref[...])
    bb = residual_t.shape[1]
    quant_t = jnp.zeros_like(residual_t)
    iota_kt = jax.lax.broadcasted_iota(jnp.int32, (K, bb), 0)
    dn = (((1,), (0,)), ((), ()))
    for l in range(NUM_LEVELS):
        r_norm = jnp.sum(residual_t * residual_t, axis=0, keepdims=True)
        d2_t = (
            r_norm
            - jax.lax.dot_general(
                cb2_ref[l], residual_t, dn,
                preferred_element_type=jnp.float32,
            )
        ) + cbnb_ref[l]
        m = jnp.min(d2_t, axis=0, keepdims=True)
        masked = jnp.where(d2_t == m, iota_kt, K)
        codes = jnp.min(masked, axis=0)
        onehot_t = (masked == codes[None, :]).astype(jnp.bfloat16)
        q_hi = jax.lax.dot_general(hi_ref[l], onehot_t, dn,
                                   preferred_element_type=jnp.float32)
        hi_bits = jax.lax.bitcast_convert_type(q_hi, jnp.int32)
        v0 = jax.lax.dot_general(b0_ref[l], onehot_t, dn,
                                 preferred_element_type=jnp.float32)
        v1 = jax.lax.dot_general(b1_ref[l], onehot_t, dn,
                                 preferred_element_type=jnp.float32)
        low16 = (v0 + v1).astype(jnp.int32)
        q_t = jax.lax.bitcast_convert_type(hi_bits | low16, jnp.float32)
        codes_ref[l, :] = codes
        quant_t = quant_t + q_t
        residual_t = residual_t - q_t
    quant_ref[...] = jnp.transpose(quant_t)


@jax.jit
def kernel(x, codebooks):
    b, d = x.shape
    cb_norms = jnp.stack(
        [jnp.sum(codebooks[l] * codebooks[l], axis=1)
         for l in range(NUM_LEVELS)], axis=0)
    cb_bytes = jax.lax.bitcast_convert_type(codebooks, jnp.uint8)
    b0_plane = cb_bytes[..., 0].astype(jnp.bfloat16).swapaxes(1, 2)
    b1_plane = (cb_bytes[..., 1].astype(jnp.float32)
                * 256.0).astype(jnp.bfloat16).swapaxes(1, 2)
    hi_plane = jax.lax.bitcast_convert_type(
        jax.lax.bitcast_convert_type(codebooks, jnp.int16)[..., 1],
        jnp.bfloat16).swapaxes(1, 2)
    grid = (b // BLOCK_B,)
    plane_spec = pl.BlockSpec((NUM_LEVELS, D, K), lambda i: (0, 0, 0))
    codes_t, quant = pl.pallas_call(
        _rq_kernel,
        grid=grid,
        in_specs=[
            pl.BlockSpec((BLOCK_B, d), lambda i: (i, 0)),
            pl.BlockSpec((NUM_LEVELS, K, D), lambda i: (0, 0, 0)),
            plane_spec,
            plane_spec,
            plane_spec,
            pl.BlockSpec((NUM_LEVELS, K), lambda i: (0, 0)),
        ],
        scratch_shapes=[
            pltpu.VMEM((NUM_LEVELS, K, BLOCK_B), jnp.float32)],
        out_specs=[
            pl.BlockSpec((NUM_LEVELS, BLOCK_B), lambda i: (0, i)),
            pl.BlockSpec((BLOCK_B, d), lambda i: (i, 0)),
        ],
        out_shape=[
            jax.ShapeDtypeStruct((NUM_LEVELS, b), jnp.int32),
            jax.ShapeDtypeStruct((b, d), jnp.float32),
        ],
    )(x, 2.0 * codebooks, hi_plane, b0_plane, b1_plane, cb_norms)
    return codes_t.T, quant


# speedup vs baseline: 1.3797x; 1.1445x over previous
import functools

import jax
import jax.numpy as jnp
from jax.experimental import pallas as pl
from jax.experimental.pallas import tpu as pltpu

NUM_LEVELS = 4
K = 1024
D = 256
BLOCK_B = 1024


def _rq_kernel(x_ref, cb2_ref, planes_ref, cbn_ref,
               codes_ref, quant_ref, cbnb_ref):
    @pl.when(pl.program_id(0) == 0)
    def _fill():
        for l in range(NUM_LEVELS):
            cbnb_ref[l] = jnp.broadcast_to(cbn_ref[l][:, None],
                                           (K, BLOCK_B))

    residual_t = jnp.transpose(x_ref[...])
    bb = residual_t.shape[1]
    quant_t = jnp.zeros_like(residual_t)
    iota_kt = jax.lax.broadcasted_iota(
        jnp.int32, (K, bb), 0).astype(jnp.float32)
    dn = (((1,), (0,)), ((), ()))
    for l in range(NUM_LEVELS):
        r_norm = jnp.sum(residual_t * residual_t, axis=0, keepdims=True)
        d2_t = (
            r_norm
            - jax.lax.dot_general(
                cb2_ref[l], residual_t, dn,
                preferred_element_type=jnp.float32,
            )
        ) + cbnb_ref[l]
        m = jnp.min(d2_t, axis=0, keepdims=True)
        masked = jnp.where(d2_t == m, iota_kt, float(K))
        codes_f = jnp.min(masked, axis=0)
        codes = codes_f.astype(jnp.int32)
        onehot_t = (masked == codes_f[None, :]).astype(jnp.bfloat16)
        g = jax.lax.dot_general(planes_ref[l], onehot_t, dn,
                                preferred_element_type=jnp.float32)
        hi_bits = jax.lax.bitcast_convert_type(g[:D], jnp.int32)
        low16 = (g[D:2 * D] + g[2 * D:]).astype(jnp.int32)
        q_t = jax.lax.bitcast_convert_type(hi_bits | low16, jnp.float32)
        codes_ref[l, :] = codes
        quant_t = quant_t + q_t
        residual_t = residual_t - q_t
    quant_ref[...] = jnp.transpose(quant_t)


@jax.jit
def kernel(x, codebooks):
    b, d = x.shape
    cb_norms = jnp.stack(
        [jnp.sum(codebooks[l] * codebooks[l], axis=1)
         for l in range(NUM_LEVELS)], axis=0)
    cb_bytes = jax.lax.bitcast_convert_type(codebooks, jnp.uint8)
    b0_plane = cb_bytes[..., 0].astype(jnp.bfloat16).swapaxes(1, 2)
    b1_plane = (cb_bytes[..., 1].astype(jnp.float32)
                * 256.0).astype(jnp.bfloat16).swapaxes(1, 2)
    hi_plane = jax.lax.bitcast_convert_type(
        jax.lax.bitcast_convert_type(codebooks, jnp.int16)[..., 1],
        jnp.bfloat16).swapaxes(1, 2)
    planes = jnp.concatenate([hi_plane, b0_plane, b1_plane], axis=1)
    grid = (b // BLOCK_B,)
    plane_spec = pl.BlockSpec((NUM_LEVELS, 3 * D, K), lambda i: (0, 0, 0))
    codes_t, quant = pl.pallas_call(
        _rq_kernel,
        grid=grid,
        in_specs=[
            pl.BlockSpec((BLOCK_B, d), lambda i: (i, 0)),
            pl.BlockSpec((NUM_LEVELS, K, D), lambda i: (0, 0, 0)),
            plane_spec,
            pl.BlockSpec((NUM_LEVELS, K), lambda i: (0, 0)),
        ],
        scratch_shapes=[
            pltpu.VMEM((NUM_LEVELS, K, BLOCK_B), jnp.float32)],
        out_specs=[
            pl.BlockSpec((NUM_LEVELS, BLOCK_B), lambda i: (0, i)),
            pl.BlockSpec((BLOCK_B, d), lambda i: (i, 0)),
        ],
        out_shape=[
            jax.ShapeDtypeStruct((NUM_LEVELS, b), jnp.int32),
            jax.ShapeDtypeStruct((b, d), jnp.float32),
        ],
    )(x, 2.0 * codebooks, planes, cb_norms)
    return codes_t.T, quant
